# Initial kernel scaffold; baseline (speedup 1.0000x reference)
import jax
import jax.numpy as jnp
from jax.experimental import pallas as pl


def kernel(h, adj, W, a):
    raise NotImplementedError("write your pallas kernel here")


# speedup vs baseline: 6809.7514x; 6809.7514x over previous
import jax
import jax.numpy as jnp
from jax.experimental import pallas as pl
from jax.experimental.pallas import tpu as pltpu

N = 2048
F = 64
ALPHA = 0.2
BR = 256


def _gat_kernel(h_ref, adj_ref, w_ref, a_ref, out_ref, hw_ref, s1_ref, s2_ref):
    @pl.when(pl.program_id(0) == 0)
    def _prologue():
        hw = jax.lax.dot_general(
            h_ref[...], w_ref[...], (((1,), (1,)), ((), ())),
            preferred_element_type=jnp.float32)
        hw_ref[...] = hw
        s1_ref[...] = jax.lax.dot_general(
            hw, a_ref[:, :F], (((1,), (1,)), ((), ())),
            preferred_element_type=jnp.float32)
        s2_ref[...] = jax.lax.dot_general(
            a_ref[:, F:], hw, (((1,), (1,)), ((), ())),
            preferred_element_type=jnp.float32)

    i = pl.program_id(0)
    adj = adj_ref[...]
    raw = s1_ref[pl.ds(i * BR, BR), :] + s2_ref[...]
    e = jnp.where(raw >= 0, raw, ALPHA * raw)
    mask = adj > 0
    m = jnp.max(jnp.where(mask, e, -jnp.inf), axis=1, keepdims=True)
    p = jnp.where(mask, jnp.exp(e - m), 0.0)
    s = jnp.sum(p, axis=1, keepdims=True)
    att = p / jnp.where(s > 0, s, 1.0)
    hp = jax.lax.dot_general(
        att, hw_ref[...], (((1,), (0,)), ((), ())),
        preferred_element_type=jnp.float32)
    out_ref[...] = jnp.where(hp > 0, hp, jnp.exp(jnp.minimum(hp, 0.0)) - 1.0)


@jax.jit
def kernel(h, adj, W, a):
    return pl.pallas_call(
        _gat_kernel,
        grid=(N // BR,),
        in_specs=[
            pl.BlockSpec((N, F), lambda i: (0, 0)),
            pl.BlockSpec((BR, N), lambda i: (i, 0)),
            pl.BlockSpec((F, F), lambda i: (0, 0)),
            pl.BlockSpec((1, 2 * F), lambda i: (0, 0)),
        ],
        out_specs=pl.BlockSpec((BR, F), lambda i: (i, 0)),
        out_shape=jax.ShapeDtypeStruct((N, F), jnp.float32),
        scratch_shapes=[
            pltpu.VMEM((N, F), jnp.float32),
            pltpu.VMEM((N, 1), jnp.float32),
            pltpu.VMEM((1, N), jnp.float32),
        ],
    )(h, adj, W, a)


# speedup vs baseline: 7573.1137x; 1.1121x over previous
import jax
import jax.numpy as jnp
from jax.experimental import pallas as pl
from jax.experimental.pallas import tpu as pltpu

N = 2048
F = 64
ALPHA = 0.2
BR = 256


def _gat_kernel(h_ref, adj_ref, w_ref, a_ref, out_ref, hwa_ref, s1_ref, s2_ref):
    @pl.when(pl.program_id(0) == 0)
    def _prologue():
        hw = jax.lax.dot_general(
            h_ref[...], w_ref[...], (((1,), (1,)), ((), ())),
            preferred_element_type=jnp.float32)
        col = jax.lax.broadcasted_iota(jnp.int32, (N, 128), 1)
        hwa_ref[...] = jnp.where(
            col < F,
            jnp.pad(hw, ((0, 0), (0, 64))),
            jnp.where(col == F, 1.0, 0.0))
        s1_ref[...] = jax.lax.dot_general(
            hw, a_ref[:, :F], (((1,), (1,)), ((), ())),
            preferred_element_type=jnp.float32)
        s2_ref[...] = jax.lax.dot_general(
            a_ref[:, F:], hw, (((1,), (1,)), ((), ())),
            preferred_element_type=jnp.float32)

    i = pl.program_id(0)
    adj = adj_ref[...]
    raw = s1_ref[pl.ds(i * BR, BR), :] + s2_ref[...]
    e = jnp.maximum(raw, ALPHA * raw)
    m = jnp.max(e * adj, axis=1, keepdims=True)
    p = adj * jnp.exp(e - m)
    mm = jax.lax.dot_general(
        p, hwa_ref[...], (((1,), (0,)), ((), ())),
        preferred_element_type=jnp.float32)
    s = mm[:, F:F + 1]
    hp = mm[:, :F] / jnp.where(s > 0, s, 1.0)
    out_ref[...] = jnp.where(hp > 0, hp, jnp.exp(jnp.minimum(hp, 0.0)) - 1.0)


@jax.jit
def kernel(h, adj, W, a):
    return pl.pallas_call(
        _gat_kernel,
        grid=(N // BR,),
        in_specs=[
            pl.BlockSpec((N, F), lambda i: (0, 0)),
            pl.BlockSpec((BR, N), lambda i: (i, 0)),
            pl.BlockSpec((F, F), lambda i: (0, 0)),
            pl.BlockSpec((1, 2 * F), lambda i: (0, 0)),
        ],
        out_specs=pl.BlockSpec((BR, F), lambda i: (i, 0)),
        out_shape=jax.ShapeDtypeStruct((N, F), jnp.float32),
        scratch_shapes=[
            pltpu.VMEM((N, 128), jnp.float32),
            pltpu.VMEM((N, 1), jnp.float32),
            pltpu.VMEM((1, N), jnp.float32),
        ],
    )(h, adj, W, a)


# speedup vs baseline: 7787.1052x; 1.0283x over previous
import jax
import jax.numpy as jnp
from jax.experimental import pallas as pl
from jax.experimental.pallas import tpu as pltpu

N = 2048
F = 64
ALPHA = 0.2
BR = 256


def _gat_kernel(h_ref, adj_ref, w_ref, a_ref, out_ref,
                hwa_ref, s1m_ref, c_ref, s2_ref):
    @pl.when(pl.program_id(0) == 0)
    def _prologue():
        hw = jax.lax.dot_general(
            h_ref[...], w_ref[...], (((1,), (1,)), ((), ())),
            preferred_element_type=jnp.float32)
        col = jax.lax.broadcasted_iota(jnp.int32, (N, 128), 1)
        hwa_ref[...] = jnp.where(
            col < F,
            jnp.pad(hw, ((0, 0), (0, 64))),
            jnp.where(col == F, 1.0, 0.0))
        s1 = jax.lax.dot_general(
            hw, a_ref[:, :F], (((1,), (1,)), ((), ())),
            preferred_element_type=jnp.float32)
        s2 = jax.lax.dot_general(
            a_ref[:, F:], hw, (((1,), (1,)), ((), ())),
            preferred_element_type=jnp.float32)
        s2_ref[...] = s2
        m2 = jnp.max(s2)
        t = s1 + m2
        m = jnp.maximum(t, ALPHA * t)
        s1m_ref[...] = s1 - m
        c_ref[...] = (ALPHA - 1.0) * m

    i = pl.program_id(0)
    adj = adj_ref[...]
    u = s1m_ref[pl.ds(i * BR, BR), :] + s2_ref[...]
    w = jnp.maximum(u, ALPHA * u + c_ref[pl.ds(i * BR, BR), :])
    p = adj * jnp.exp(w)
    mm = jax.lax.dot_general(
        p, hwa_ref[...], (((1,), (0,)), ((), ())),
        preferred_element_type=jnp.float32)
    s = mm[:, F:F + 1]
    hp = mm[:, :F] / jnp.where(s > 0, s, 1.0)
    out_ref[...] = jnp.where(hp > 0, hp, jnp.exp(jnp.minimum(hp, 0.0)) - 1.0)


@jax.jit
def kernel(h, adj, W, a):
    return pl.pallas_call(
        _gat_kernel,
        grid=(N // BR,),
        in_specs=[
            pl.BlockSpec((N, F), lambda i: (0, 0)),
            pl.BlockSpec((BR, N), lambda i: (i, 0)),
            pl.BlockSpec((F, F), lambda i: (0, 0)),
            pl.BlockSpec((1, 2 * F), lambda i: (0, 0)),
        ],
        out_specs=pl.BlockSpec((BR, F), lambda i: (i, 0)),
        out_shape=jax.ShapeDtypeStruct((N, F), jnp.float32),
        scratch_shapes=[
            pltpu.VMEM((N, 128), jnp.float32),
            pltpu.VMEM((N, 1), jnp.float32),
            pltpu.VMEM((N, 1), jnp.float32),
            pltpu.VMEM((1, N), jnp.float32),
        ],
    )(h, adj, W, a)
